# Initial kernel scaffold; baseline (speedup 1.0000x reference)
import jax
import jax.numpy as jnp
from jax.experimental import pallas as pl


def kernel(neighboe_xyz, center_xyz, W1c, b1c, gc, bc, W2c, b2c, W1p, b1p, gp, bp, W2p, b2p, Wac, bac, Wap, bap, gamma, Wg1, bg1, gg, bg, Wg2, bg2):
    raise NotImplementedError("write your pallas kernel here")


# speedup vs baseline: 6.0250x; 6.0250x over previous
import math
from functools import partial

import jax
import jax.numpy as jnp
from jax import lax
from jax.experimental import pallas as pl

_EPS = 1e-06
_RB = 2
_AB = 12
_NB = _RB * _AB
_C = 32


def _features(nb, ct):
    d = nb - ct[:, None, :]
    d0, d1, d2 = d[0], d[1], d[2]
    r2xy = d0 * d0 + d1 * d1
    euc = jnp.sqrt(r2xy + d2 * d2)
    r = jnp.maximum(euc, _EPS)
    rho = jnp.maximum(jnp.sqrt(r2xy), _EPS)
    theta = jnp.arctan2(d1, d0)
    phi = jnp.arctan2(d2, rho)
    r_mean = jnp.mean(r, axis=0, keepdims=True)
    r_norm = jnp.minimum(r / (r_mean + _EPS), 3.0) / 3.0
    r_idx = r_norm * (_RB - 1e-06)
    a_unit = (theta + math.pi) / (2.0 * math.pi)
    a_idx = a_unit * (_AB - 1e-06)
    return [d0, d1, d2, euc,
            r_norm, jnp.sin(theta), jnp.cos(theta), jnp.sin(phi), jnp.cos(phi),
            r_idx, a_idx]


def _bilin(idx, maxb):
    i0f = jnp.floor(idx)
    w1 = idx - i0f
    w0 = 1.0 - w1
    i0 = jnp.clip(i0f, 0.0, maxb - 1.0)
    i1 = jnp.clip(i0f + 1.0, 0.0, maxb - 1.0)
    return i0, i1, w0, w1


def _flat(x3):
    f, k, tn = x3.shape
    return x3.reshape(f, k * tn)


def _splat_weights(r_idx, a_idx):
    ri0, ri1, rw0, rw1 = _bilin(r_idx, float(_RB))
    ai0, ai1, aw0, aw1 = _bilin(a_idx, float(_AB))
    p = r_idx.shape[1]
    iota = lax.broadcasted_iota(jnp.int32, (_NB, p), 0)
    w2 = jnp.zeros((_NB, p), jnp.float32)
    corners = []
    for ri, ai, wt in ((ri0, ai0, rw0 * aw0), (ri0, ai1, rw0 * aw1),
                       (ri1, ai0, rw1 * aw0), (ri1, ai1, rw1 * aw1)):
        fb = (ri * float(_AB) + ai).astype(jnp.int32)
        corners.append((fb, wt))
        w2 = w2 + jnp.where(iota == fb, wt, 0.0)
    return w2, corners


def _bn_relu(y, vec, j):
    g = vec[:, j + 1:j + 2]
    m = vec[:, j + 2:j + 3]
    q = vec[:, j + 3:j + 4]
    b = vec[:, j + 4:j + 5]
    return jnp.maximum(g * (y - m) / q + b, 0.0)


def _fused_tile(nb, ct, w1c, w1p, wmat, vec, K, TN):
    feats = _features(nb, ct)
    f11 = _flat(jnp.stack(feats, axis=0))

    y1c = jnp.dot(w1c, f11[0:4], preferred_element_type=jnp.float32,
                    precision=lax.Precision.HIGHEST) + vec[:, 0:1]
    h_c = _bn_relu(y1c, vec, 0)
    c_raw = jnp.dot(wmat[:, 0:32], h_c,
                    preferred_element_type=jnp.float32,
                    precision=lax.Precision.HIGHEST) + vec[:, 5:6]

    y1p = jnp.dot(w1p, f11[4:9], preferred_element_type=jnp.float32,
                    precision=lax.Precision.HIGHEST) + vec[:, 6:7]
    h_p = _bn_relu(y1p, vec, 6)
    p_raw = jnp.dot(wmat[:, 32:64], h_p,
                    preferred_element_type=jnp.float32,
                    precision=lax.Precision.HIGHEST) + vec[:, 11:12]

    w2, corners = _splat_weights(f11[9:10], f11[10:11])

    grid = jnp.zeros((_C, _NB, TN), jnp.float32)
    cnt = jnp.zeros((_NB, TN), jnp.float32)
    for k in range(K):
        sl = slice(k * TN, (k + 1) * TN)
        grid = grid + p_raw[:, sl][:, None, :] * w2[:, sl][None, :, :]
        cnt = cnt + w2[:, sl]
    recip = 1.0 / jnp.maximum(cnt, _EPS)

    iota_t = lax.broadcasted_iota(jnp.int32, (_NB, TN), 0)
    vals = []
    for fb, wt in corners:
        m = jnp.where(iota_t == fb[:, 0:TN], recip, 0.0)
        vals.append((jnp.sum(grid * m[None, :, :], axis=1), wt))
    outs = []
    for k in range(K):
        sl = slice(k * TN, (k + 1) * TN)
        o = vals[0][0] * vals[0][1][:, sl]
        for v, wt in vals[1:]:
            o = o + v * wt[:, sl]
        outs.append(o)
    p_al = jnp.concatenate(outs, axis=1)

    fused = (jnp.dot(wmat[:, 64:96], c_raw,
                     preferred_element_type=jnp.float32,
                    precision=lax.Precision.HIGHEST) + vec[:, 12:13]) \
        + vec[:, 14:15] * (jnp.dot(wmat[:, 96:128], p_al,
                                   preferred_element_type=jnp.float32,
                                   precision=lax.Precision.HIGHEST)
                           + vec[:, 13:14])
    return fused


def _first(s_ref, val):
    init = (pl.program_id(0) == 0) & (pl.program_id(1) == 0)

    @pl.when(init)
    def _():
        s_ref[...] = val

    @pl.when(jnp.logical_not(init))
    def _():
        s_ref[...] = s_ref[...] + val


def _p1_body(nb_ref, ct_ref, s_ref, *, K, TN):
    feats = _features(nb_ref[0], ct_ref[0])
    f9 = _flat(jnp.stack(feats[:9], axis=0))
    f10 = jnp.concatenate([f9, jnp.ones((1, K * TN), jnp.float32)], axis=0)
    s = lax.dot_general(f10, f10, (((1,), (1,)), ((), ())),
                        preferred_element_type=jnp.float32,
                        precision=lax.Precision.HIGHEST)
    _first(s_ref, s)


def _p2_body(nb_ref, ct_ref, w1c_ref, w1p_ref, wmat_ref, vec_ref,
             s_ref, *, K, TN):
    fused = _fused_tile(nb_ref[0], ct_ref[0], w1c_ref[...], w1p_ref[...],
                        wmat_ref[...], vec_ref[...], K, TN)
    f33 = jnp.concatenate([fused, jnp.ones((1, K * TN), jnp.float32)], axis=0)
    s = lax.dot_general(f33, f33, (((1,), (1,)), ((), ())),
                        preferred_element_type=jnp.float32,
                        precision=lax.Precision.HIGHEST)
    _first(s_ref, s)


def _p3_body(nb_ref, ct_ref, w1c_ref, w1p_ref, wmat_ref, vec_ref,
             wg_ref, vg_ref, out_ref, *, K, TN):
    fused = _fused_tile(nb_ref[0], ct_ref[0], w1c_ref[...], w1p_ref[...],
                        wmat_ref[...], vec_ref[...], K, TN)
    y = jnp.dot(wg_ref[:, 0:32], fused,
                preferred_element_type=jnp.float32,
                precision=lax.Precision.HIGHEST) + vg_ref[:, 0:1]
    h = _bn_relu(y, vg_ref[...], 0)
    o = jnp.dot(wg_ref[:, 32:64], h,
                preferred_element_type=jnp.float32,
                precision=lax.Precision.HIGHEST) + vg_ref[:, 5:6]
    o3 = o.reshape(_C, K, TN)
    out_ref[0] = jnp.swapaxes(o3, 1, 2)


def _bn_stats(W, b, mu, cov):
    m = jnp.dot(W, mu, precision=lax.Precision.HIGHEST) + b
    v = jnp.sum(jnp.dot(W, cov, precision=lax.Precision.HIGHEST) * W, axis=1)
    return m, jnp.sqrt(v + 1e-05)


def kernel(neighboe_xyz, center_xyz, W1c, b1c, gc, bc, W2c, b2c, W1p, b1p,
           gp, bp, W2p, b2p, Wac, bac, Wap, bap, gamma, Wg1, bg1, gg, bg,
           Wg2, bg2):
    B, _, N, K = neighboe_xyz.shape
    nbT = jnp.transpose(neighboe_xyz, (0, 1, 3, 2))

    TN1 = 512 if N % 512 == 0 else N
    TN2 = 256 if N % 256 == 0 else N
    cnt_total = float(B * N * K)

    nb_spec = lambda TN: pl.BlockSpec((1, 3, K, TN), lambda b, t: (b, 0, 0, t))
    ct_spec = lambda TN: pl.BlockSpec((1, 3, TN), lambda b, t: (b, 0, t))
    w_spec = lambda *shp: pl.BlockSpec(shp, lambda b, t: tuple(0 for _ in shp))

    s10 = pl.pallas_call(
        partial(_p1_body, K=K, TN=TN1),
        grid=(B, N // TN1),
        in_specs=[nb_spec(TN1), ct_spec(TN1)],
        out_specs=w_spec(10, 10),
        out_shape=jax.ShapeDtypeStruct((10, 10), jnp.float32),
    )(nbT, center_xyz)

    mu9 = s10[:9, 9] / cnt_total
    cov9 = s10[:9, :9] / cnt_total - mu9[:, None] * mu9[None, :]

    m1c, q1c = _bn_stats(W1c, b1c, mu9[:4], cov9[:4, :4])
    m1p, q1p = _bn_stats(W1p, b1p, mu9[4:9], cov9[4:9, 4:9])
    g0col = jnp.full((_C,), gamma[0], jnp.float32)
    vec2 = jnp.stack([b1c, gc, m1c, q1c, bc, b2c,
                      b1p, gp, m1p, q1p, bp, b2p,
                      bac, bap, g0col], axis=1)
    wmat2 = jnp.concatenate([W2c, W2p, Wac, Wap], axis=1)

    s33 = pl.pallas_call(
        partial(_p2_body, K=K, TN=TN2),
        grid=(B, N // TN2),
        in_specs=[nb_spec(TN2), ct_spec(TN2), w_spec(32, 4), w_spec(32, 5),
                  w_spec(32, 128), w_spec(32, 15)],
        out_specs=w_spec(33, 33),
        out_shape=jax.ShapeDtypeStruct((33, 33), jnp.float32),
    )(nbT, center_xyz, W1c, W1p, wmat2, vec2)

    muf = s33[:32, 32] / cnt_total
    covf = s33[:32, :32] / cnt_total - muf[:, None] * muf[None, :]
    mg, qg = _bn_stats(Wg1, bg1, muf, covf)
    vec3 = jnp.stack([bg1, gg, mg, qg, bg, bg2], axis=1)
    wmat3 = jnp.concatenate([Wg1, Wg2], axis=1)

    out = pl.pallas_call(
        partial(_p3_body, K=K, TN=TN2),
        grid=(B, N // TN2),
        in_specs=[nb_spec(TN2), ct_spec(TN2), w_spec(32, 4), w_spec(32, 5),
                  w_spec(32, 128), w_spec(32, 15),
                  w_spec(32, 64), w_spec(32, 6)],
        out_specs=pl.BlockSpec((1, _C, TN2, K), lambda b, t: (b, 0, t, 0)),
        out_shape=jax.ShapeDtypeStruct((B, _C, N, K), jnp.float32),
    )(nbT, center_xyz, W1c, W1p, wmat2, vec2, wmat3, vec3)

    return out
